# baseline (speedup 1.0000x reference)
import functools
import math

import jax
import jax.numpy as jnp
from jax import lax
from jax.experimental import pallas as pl
from jax.experimental.pallas import tpu as pltpu
from jax.experimental.pallas import tpu_sc as plsc

_K = 50000
_DS = 256
_DA = 64
_BK = 2000
_NB = _K // _BK
_TH = 1000
_RS = 5.0
_BW = 5.0 * _TH / math.sqrt(_DS + _DA)
_EW = 1.0 / _K
_UNITS = (1.0 / _TH - 1e-6) * _K
_TOPK = 50
_ROUNDS = 50


def _stats_body(es_ref, ea_ref, ss_ref, sq_ref, as_ref, aq_ref):
    i = pl.program_id(0)

    @pl.when(i == 0)
    def _():
        ss_ref[...] = jnp.zeros_like(ss_ref)
        sq_ref[...] = jnp.zeros_like(sq_ref)
        as_ref[...] = jnp.zeros_like(as_ref)
        aq_ref[...] = jnp.zeros_like(aq_ref)

    es = es_ref[...]
    ea = ea_ref[...]
    ss_ref[...] += jnp.sum(es, axis=0, keepdims=True)
    sq_ref[...] += jnp.sum(es * es, axis=0, keepdims=True)
    as_ref[...] += jnp.sum(ea, axis=0, keepdims=True)
    aq_ref[...] += jnp.sum(ea * ea, axis=0, keepdims=True)


def _dist_body(es_ref, ea_ref, s_ref, a_ref, ss_ref, sq_ref, as_ref, aq_ref,
               d_ref):
    n = jnp.float32(_K)
    ms = ss_ref[...] / n
    vs = jnp.maximum(sq_ref[...] / n - ms * ms, 0.0)
    inv_s = 1.0 / (jnp.sqrt(vs) + 1e-8)
    ma = as_ref[...] / n
    va = jnp.maximum(aq_ref[...] / n - ma * ma, 0.0)
    inv_a = 1.0 / (jnp.sqrt(va) + 1e-8)
    ds = (es_ref[...] - s_ref[...]) * inv_s
    da = (ea_ref[...] - a_ref[...]) * inv_a
    d2 = (jnp.dot(ds * ds, jnp.ones((_DS, 1), jnp.float32),
                  preferred_element_type=jnp.float32)
          + jnp.dot(da * da, jnp.ones((_DA, 1), jnp.float32),
                    preferred_element_type=jnp.float32))
    d_ref[...] = jnp.sqrt(d2)


_NW = 32
_CH = 1568
_NV = _CH // 16


def _fold(vals, op):
    while len(vals) > 1:
        vals = [op(vals[i], vals[i + 1]) for i in range(0, len(vals) - 1, 2)] \
            + ([vals[-1]] if len(vals) % 2 else [])
    return vals[0]


def _hmax(vec):
    return _fold([vec[i] for i in range(16)], jnp.maximum)


_sc_mesh = plsc.VectorSubcoreMesh(core_axis_name="c", subcore_axis_name="s")


@functools.partial(
    pl.kernel,
    out_type=jax.ShapeDtypeStruct((_NW * _CH,), jnp.float32),
    mesh=_sc_mesh,
    scratch_types=[pltpu.VMEM((_CH,), jnp.float32)],
)
def _sc_filter(d_hbm, filt_hbm, d_v):
    wid = lax.axis_index("s") * 2 + lax.axis_index("c")
    start = jnp.where(wid == _NW - 1, _K - _CH, wid * _CH)
    pltpu.sync_copy(d_hbm.at[pl.ds(start, _CH)], d_v)
    inf16 = jnp.full((16,), jnp.inf, jnp.float32)

    flag = jnp.where(wid == _NW - 1, jnp.inf, 0.0).astype(jnp.float32)
    flag16 = jnp.full((16,), flag, jnp.float32)
    for j in range(11):
        d_v[pl.ds(16 * j, 16)] = d_v[pl.ds(16 * j, 16)] + flag16

    m_prev = jnp.full((16,), -jnp.inf, jnp.float32)
    for _ in range(4):
        def pass_fn(j, acc, mp=m_prev):
            v = d_v[pl.ds(16 * j, 16)]
            return jnp.minimum(acc, jnp.where(v > mp, v, jnp.inf))

        m_prev = lax.fori_loop(0, _NV, pass_fn, inf16)

    t16 = jnp.full((16,), _hmax(m_prev), jnp.float32)

    def filt_fn(j, carry):
        v = d_v[pl.ds(16 * j, 16)]
        d_v[pl.ds(16 * j, 16)] = jnp.where(v <= t16, v, jnp.inf)
        return carry

    lax.fori_loop(0, _NV, filt_fn, jnp.int32(0))
    pltpu.sync_copy(d_v, filt_hbm.at[pl.ds(wid * _CH, _CH)])


def _merge_body(c_ref, r_ref):
    filt = c_ref[...]
    bits = lax.bitcast_convert_type(filt, jnp.int32)

    def step(_, lohi):
        lo, hi = lohi
        mid = lo + (hi - lo) // 2
        cnt = jnp.sum(jnp.where(bits <= mid, 1.0, 0.0))
        big = cnt >= jnp.float32(_TOPK)
        return jnp.where(big, lo, mid + 1), jnp.where(big, mid, hi)

    tb, _ = lax.fori_loop(
        0, 31, step, (jnp.int32(0), jnp.int32(0x7F800000)))
    t = jnp.min(jnp.where(bits >= tb, filt, jnp.inf))
    s = jnp.sum(jnp.where(filt < t, filt, 0.0))
    c = jnp.sum(jnp.where(filt < t, 1.0, 0.0))
    cost = (s + (jnp.float32(_UNITS) - c) * t) * jnp.float32(_EW)
    reward = jnp.float32(_RS) * jnp.exp(jnp.float32(-_BW) * cost)
    r_ref[...] = jnp.reshape(reward, (1, 1))


def kernel(state, action, expert_states, expert_actions):
    f32 = jnp.float32
    ss, sq, as_, aq = pl.pallas_call(
        _stats_body,
        grid=(_NB,),
        in_specs=[
            pl.BlockSpec((_BK, _DS), lambda i: (i, 0)),
            pl.BlockSpec((_BK, _DA), lambda i: (i, 0)),
        ],
        out_specs=[
            pl.BlockSpec((1, _DS), lambda i: (0, 0)),
            pl.BlockSpec((1, _DS), lambda i: (0, 0)),
            pl.BlockSpec((1, _DA), lambda i: (0, 0)),
            pl.BlockSpec((1, _DA), lambda i: (0, 0)),
        ],
        out_shape=[
            jax.ShapeDtypeStruct((1, _DS), f32),
            jax.ShapeDtypeStruct((1, _DS), f32),
            jax.ShapeDtypeStruct((1, _DA), f32),
            jax.ShapeDtypeStruct((1, _DA), f32),
        ],
    )(expert_states, expert_actions)

    d = pl.pallas_call(
        _dist_body,
        grid=(_NB,),
        in_specs=[
            pl.BlockSpec((_BK, _DS), lambda i: (i, 0)),
            pl.BlockSpec((_BK, _DA), lambda i: (i, 0)),
            pl.BlockSpec((1, _DS), lambda i: (0, 0)),
            pl.BlockSpec((1, _DA), lambda i: (0, 0)),
            pl.BlockSpec((1, _DS), lambda i: (0, 0)),
            pl.BlockSpec((1, _DS), lambda i: (0, 0)),
            pl.BlockSpec((1, _DA), lambda i: (0, 0)),
            pl.BlockSpec((1, _DA), lambda i: (0, 0)),
        ],
        out_specs=pl.BlockSpec((_BK, 1), lambda i: (i, 0)),
        out_shape=jax.ShapeDtypeStruct((_K, 1), f32),
    )(expert_states, expert_actions, state, action, ss, sq, as_, aq)

    filt = _sc_filter(d.reshape(_K))

    r = pl.pallas_call(
        _merge_body,
        out_shape=jax.ShapeDtypeStruct((1, 1), f32),
    )(filt.reshape(_NW * _CH // 128, 128))
    return r[0, 0]


# speedup vs baseline: 1.1056x; 1.1056x over previous
import functools
import math

import jax
import jax.numpy as jnp
from jax import lax
from jax.experimental import pallas as pl
from jax.experimental.pallas import tpu as pltpu
from jax.experimental.pallas import tpu_sc as plsc

_K = 50000
_DS = 256
_DA = 64
_BK = 5000
_NB = _K // _BK
_TH = 1000
_RS = 5.0
_BW = 5.0 * _TH / math.sqrt(_DS + _DA)
_EW = 1.0 / _K
_UNITS = (1.0 / _TH - 1e-6) * _K
_TOPK = 50
_ROUNDS = 50


def _stats_body(es_ref, ea_ref, ss_ref, sq_ref, as_ref, aq_ref):
    i = pl.program_id(0)

    @pl.when(i == 0)
    def _():
        ss_ref[...] = jnp.zeros_like(ss_ref)
        sq_ref[...] = jnp.zeros_like(sq_ref)
        as_ref[...] = jnp.zeros_like(as_ref)
        aq_ref[...] = jnp.zeros_like(aq_ref)

    es = es_ref[...]
    ea = ea_ref[...]
    ss_ref[...] += jnp.sum(es, axis=0, keepdims=True)
    sq_ref[...] += jnp.sum(es * es, axis=0, keepdims=True)
    as_ref[...] += jnp.sum(ea, axis=0, keepdims=True)
    aq_ref[...] += jnp.sum(ea * ea, axis=0, keepdims=True)


def _dist_body(es_ref, ea_ref, s_ref, a_ref, ss_ref, sq_ref, as_ref, aq_ref,
               d_ref):
    n = jnp.float32(_K)
    ms = ss_ref[...] / n
    vs = jnp.maximum(sq_ref[...] / n - ms * ms, 0.0)
    inv_s = 1.0 / (jnp.sqrt(vs) + 1e-8)
    ma = as_ref[...] / n
    va = jnp.maximum(aq_ref[...] / n - ma * ma, 0.0)
    inv_a = 1.0 / (jnp.sqrt(va) + 1e-8)
    ds = (es_ref[...] - s_ref[...]) * inv_s
    da = (ea_ref[...] - a_ref[...]) * inv_a
    d2 = (jnp.dot(ds * ds, jnp.ones((_DS, 1), jnp.float32),
                  preferred_element_type=jnp.float32)
          + jnp.dot(da * da, jnp.ones((_DA, 1), jnp.float32),
                    preferred_element_type=jnp.float32))
    d_ref[...] = jnp.sqrt(d2)


_NW = 32
_CH = 1568
_NV = _CH // 16


def _fold(vals, op):
    while len(vals) > 1:
        vals = [op(vals[i], vals[i + 1]) for i in range(0, len(vals) - 1, 2)] \
            + ([vals[-1]] if len(vals) % 2 else [])
    return vals[0]


def _hmax(vec):
    return _fold([vec[i] for i in range(16)], jnp.maximum)


_sc_mesh = plsc.VectorSubcoreMesh(core_axis_name="c", subcore_axis_name="s")


@functools.partial(
    pl.kernel,
    out_type=jax.ShapeDtypeStruct((_NW * _CH,), jnp.float32),
    mesh=_sc_mesh,
    scratch_types=[pltpu.VMEM((_CH,), jnp.float32)],
)
def _sc_filter(d_hbm, filt_hbm, d_v):
    wid = lax.axis_index("s") * 2 + lax.axis_index("c")
    start = jnp.where(wid == _NW - 1, _K - _CH, wid * _CH)
    pltpu.sync_copy(d_hbm.at[pl.ds(start, _CH)], d_v)
    inf16 = jnp.full((16,), jnp.inf, jnp.float32)

    flag = jnp.where(wid == _NW - 1, jnp.inf, 0.0).astype(jnp.float32)
    flag16 = jnp.full((16,), flag, jnp.float32)
    for j in range(11):
        d_v[pl.ds(16 * j, 16)] = d_v[pl.ds(16 * j, 16)] + flag16

    m_prev = jnp.full((16,), -jnp.inf, jnp.float32)
    for _ in range(4):
        def pass_fn(j, acc, mp=m_prev):
            v = d_v[pl.ds(16 * j, 16)]
            return jnp.minimum(acc, jnp.where(v > mp, v, jnp.inf))

        m_prev = lax.fori_loop(0, _NV, pass_fn, inf16)

    t16 = jnp.full((16,), _hmax(m_prev), jnp.float32)

    def filt_fn(j, carry):
        v = d_v[pl.ds(16 * j, 16)]
        d_v[pl.ds(16 * j, 16)] = jnp.where(v <= t16, v, jnp.inf)
        return carry

    lax.fori_loop(0, _NV, filt_fn, jnp.int32(0))
    pltpu.sync_copy(d_v, filt_hbm.at[pl.ds(wid * _CH, _CH)])


def _merge_body(c_ref, r_ref):
    filt = c_ref[...]
    bits = lax.bitcast_convert_type(filt, jnp.int32)

    def step(_, lohi):
        lo, hi = lohi
        mid = lo + (hi - lo) // 2
        cnt = jnp.sum(jnp.where(bits <= mid, 1.0, 0.0))
        big = cnt >= jnp.float32(_TOPK)
        return jnp.where(big, lo, mid + 1), jnp.where(big, mid, hi)

    tb, _ = lax.fori_loop(
        0, 31, step, (jnp.int32(0), jnp.int32(0x7F800000)))
    t = jnp.min(jnp.where(bits >= tb, filt, jnp.inf))
    s = jnp.sum(jnp.where(filt < t, filt, 0.0))
    c = jnp.sum(jnp.where(filt < t, 1.0, 0.0))
    cost = (s + (jnp.float32(_UNITS) - c) * t) * jnp.float32(_EW)
    reward = jnp.float32(_RS) * jnp.exp(jnp.float32(-_BW) * cost)
    r_ref[...] = jnp.reshape(reward, (1, 1))


def kernel(state, action, expert_states, expert_actions):
    f32 = jnp.float32
    ss, sq, as_, aq = pl.pallas_call(
        _stats_body,
        grid=(_NB,),
        in_specs=[
            pl.BlockSpec((_BK, _DS), lambda i: (i, 0)),
            pl.BlockSpec((_BK, _DA), lambda i: (i, 0)),
        ],
        out_specs=[
            pl.BlockSpec((1, _DS), lambda i: (0, 0)),
            pl.BlockSpec((1, _DS), lambda i: (0, 0)),
            pl.BlockSpec((1, _DA), lambda i: (0, 0)),
            pl.BlockSpec((1, _DA), lambda i: (0, 0)),
        ],
        out_shape=[
            jax.ShapeDtypeStruct((1, _DS), f32),
            jax.ShapeDtypeStruct((1, _DS), f32),
            jax.ShapeDtypeStruct((1, _DA), f32),
            jax.ShapeDtypeStruct((1, _DA), f32),
        ],
    )(expert_states, expert_actions)

    d = pl.pallas_call(
        _dist_body,
        grid=(_NB,),
        in_specs=[
            pl.BlockSpec((_BK, _DS), lambda i: (i, 0)),
            pl.BlockSpec((_BK, _DA), lambda i: (i, 0)),
            pl.BlockSpec((1, _DS), lambda i: (0, 0)),
            pl.BlockSpec((1, _DA), lambda i: (0, 0)),
            pl.BlockSpec((1, _DS), lambda i: (0, 0)),
            pl.BlockSpec((1, _DS), lambda i: (0, 0)),
            pl.BlockSpec((1, _DA), lambda i: (0, 0)),
            pl.BlockSpec((1, _DA), lambda i: (0, 0)),
        ],
        out_specs=pl.BlockSpec((_BK, 1), lambda i: (i, 0)),
        out_shape=jax.ShapeDtypeStruct((_K, 1), f32),
    )(expert_states, expert_actions, state, action, ss, sq, as_, aq)

    filt = _sc_filter(d.reshape(_K))

    r = pl.pallas_call(
        _merge_body,
        out_shape=jax.ShapeDtypeStruct((1, 1), f32),
    )(filt.reshape(_NW * _CH // 128, 128))
    return r[0, 0]
